# Initial kernel scaffold; baseline (speedup 1.0000x reference)
import jax
import jax.numpy as jnp
from jax.experimental import pallas as pl


def kernel(x, c_2, u_2, batch, We, be, conv_W1, conv_b1, conv_W2, conv_b2, Wr1, br1, Wr2, br2):
    raise NotImplementedError("write your pallas kernel here")


# speedup vs baseline: 4.2516x; 4.2516x over previous
import functools

import jax
import jax.numpy as jnp
from jax import lax
from jax.experimental import pallas as pl
from jax.experimental.pallas import tpu as pltpu
from jax.experimental.pallas import tpu_sc as plsc


_DOT = functools.partial(
    lax.dot_general, precision=jax.lax.Precision.HIGHEST,
    preferred_element_type=jnp.float32)


def _mm(a, b):
    return _DOT(a, b, (((a.ndim - 1,), (0,)), ((), ())))


_ROW_BLK = 1000


def _encoder_body(x_ref, we_ref, be_ref, hl_ref, hr_ref):
    h = _mm(x_ref[...], we_ref[...]) + be_ref[...]
    hl_ref[...] = h[:, :128]
    hr_ref[...] = h[:, 128:]


def _encoder(x, We, be):
    n, k = x.shape
    d = We.shape[1]
    grid = (n // _ROW_BLK,)
    return pl.pallas_call(
        _encoder_body,
        grid=grid,
        in_specs=[
            pl.BlockSpec((_ROW_BLK, k), lambda i: (i, 0)),
            pl.BlockSpec((k, d), lambda i: (0, 0)),
            pl.BlockSpec((1, d), lambda i: (0, 0)),
        ],
        out_specs=[
            pl.BlockSpec((_ROW_BLK, d // 2), lambda i: (i, 0)),
            pl.BlockSpec((_ROW_BLK, d // 2), lambda i: (i, 0)),
        ],
        out_shape=[
            jax.ShapeDtypeStruct((n, d // 2), jnp.float32),
            jax.ShapeDtypeStruct((n, d // 2), jnp.float32),
        ],
    )(x, We, be.reshape(1, d))


def _mlp_body(zl_ref, zr_ref, w1t_ref, w1b_ref, b1_ref, w2l_ref, w2r_ref,
              b2l_ref, b2r_ref, hl_ref, hr_ref):
    a = _mm(zl_ref[...], w1t_ref[...]) + _mm(zr_ref[...], w1b_ref[...])
    a = jnp.maximum(a + b1_ref[...], 0.0)
    hl_ref[...] = jnp.maximum(_mm(a, w2l_ref[...]) + b2l_ref[...], 0.0)
    hr_ref[...] = jnp.maximum(_mm(a, w2r_ref[...]) + b2r_ref[...], 0.0)


def _mlp(zL, zR, W1, b1, W2, b2):
    n, dh = zL.shape
    d = 2 * dh
    grid = (n // _ROW_BLK,)
    full = lambda r, c: pl.BlockSpec((r, c), lambda i: (0, 0))
    return pl.pallas_call(
        _mlp_body,
        grid=grid,
        in_specs=[
            pl.BlockSpec((_ROW_BLK, dh), lambda i: (i, 0)),
            pl.BlockSpec((_ROW_BLK, dh), lambda i: (i, 0)),
            full(dh, d), full(dh, d), full(1, d),
            full(d, dh), full(d, dh), full(1, dh), full(1, dh),
        ],
        out_specs=[
            pl.BlockSpec((_ROW_BLK, dh), lambda i: (i, 0)),
            pl.BlockSpec((_ROW_BLK, dh), lambda i: (i, 0)),
        ],
        out_shape=[
            jax.ShapeDtypeStruct((n, dh), jnp.float32),
            jax.ShapeDtypeStruct((n, dh), jnp.float32),
        ],
    )(zL, zR, W1[:dh], W1[dh:], b1.reshape(1, d),
      W2[:, :dh], W2[:, dh:], b2[:dh].reshape(1, dh), b2[dh:].reshape(1, dh))


def _readout_body(hl_ref, hr_ref, b_ref, w1t_ref, w1b_ref, b1_ref, w2_ref,
                  b2_ref, out_ref, accl, accr):
    i = pl.program_id(0)
    ng = accl.shape[0]

    @pl.when(i == 0)
    def _():
        accl[...] = jnp.zeros_like(accl)
        accr[...] = jnp.zeros_like(accr)

    gids = b_ref[0, 0, :]
    onehot = (lax.broadcasted_iota(jnp.int32, (ng, gids.shape[0]), 0)
              == gids[None, :]).astype(jnp.float32)
    accl[...] += _mm(onehot, hl_ref[...])
    accr[...] += _mm(onehot, hr_ref[...])

    @pl.when(i == pl.num_programs(0) - 1)
    def _():
        a = _mm(accl[...], w1t_ref[...]) + _mm(accr[...], w1b_ref[...])
        a = jnp.maximum(a + b1_ref[...], 0.0)
        out_ref[...] = _mm(a, w2_ref[...]) + b2_ref[...]


def _readout(hL, hR, batch, Wr1, br1, Wr2, br2, num_graphs):
    n, dh = hL.shape
    d = 2 * dh
    nc = Wr2.shape[1]
    grid = (n // _ROW_BLK,)
    b3 = batch.reshape(n // _ROW_BLK, 1, _ROW_BLK)
    full = lambda r, c: pl.BlockSpec((r, c), lambda i: (0, 0))
    return pl.pallas_call(
        _readout_body,
        grid=grid,
        in_specs=[
            pl.BlockSpec((_ROW_BLK, dh), lambda i: (i, 0)),
            pl.BlockSpec((_ROW_BLK, dh), lambda i: (i, 0)),
            pl.BlockSpec((1, 1, _ROW_BLK), lambda i: (i, 0, 0)),
            full(dh, d), full(dh, d), full(1, d),
            full(d, nc), full(1, nc),
        ],
        out_specs=pl.BlockSpec((num_graphs, nc), lambda i: (0, 0)),
        out_shape=jax.ShapeDtypeStruct((num_graphs, nc), jnp.float32),
        scratch_shapes=[
            pltpu.VMEM((num_graphs, dh), jnp.float32),
            pltpu.VMEM((num_graphs, dh), jnp.float32),
        ],
    )(hL, hR, b3, Wr1[:dh], Wr1[dh:], br1.reshape(1, d), Wr2,
      br2.reshape(1, nc))


_EDGE_BLK = 128
_N_TILES = 16


def _sc_agg(hL, hR, c2, u2):
    n, dh = hL.shape
    e = c2.shape[0]
    n_blocks = e // _EDGE_BLK
    stripe = (n // _N_TILES) // 8 * 8
    tail = n - stripe * _N_TILES
    mesh = plsc.VectorSubcoreMesh(core_axis_name="c", subcore_axis_name="s")

    @functools.partial(
        pl.kernel,
        out_type=(jax.ShapeDtypeStruct((n, dh), jnp.float32),
                  jax.ShapeDtypeStruct((n, dh), jnp.float32)),
        mesh=mesh,
        scratch_types=[
            pltpu.VMEM_SHARED((n, dh), jnp.float32),
            pltpu.VMEM((_EDGE_BLK,), jnp.int32),
            pltpu.VMEM((_EDGE_BLK,), jnp.int32),
            pltpu.VMEM((_EDGE_BLK, dh), jnp.float32),
            pltpu.SemaphoreType.DMA,
        ],
    )
    def agg(hl_hbm, hr_hbm, c2_hbm, u2_hbm, zl_hbm, zr_hbm,
            acc_sh, cidx, uidx, rows, sem):
        cid = lax.axis_index("c")
        sid = lax.axis_index("s")

        def run(tab_hbm, out_hbm):
            r0 = pl.multiple_of(sid * stripe, 8)
            pltpu.sync_copy(tab_hbm.at[pl.ds(r0, stripe)],
                            acc_sh.at[pl.ds(r0, stripe)])
            if tail:
                @pl.when(sid == _N_TILES - 1)
                def _():
                    pltpu.sync_copy(tab_hbm.at[pl.ds(stripe * _N_TILES, tail)],
                                    acc_sh.at[pl.ds(stripe * _N_TILES, tail)])
            plsc.subcore_barrier()

            base_blocks = n_blocks // _N_TILES
            extra = n_blocks - base_blocks * _N_TILES
            nk = base_blocks + jnp.where(sid < extra, 1, 0)

            @pl.loop(0, nk)
            def _(k):
                e0 = (sid + k * _N_TILES) * _EDGE_BLK
                pltpu.sync_copy(c2_hbm.at[pl.ds(e0, _EDGE_BLK)], cidx)
                pltpu.sync_copy(u2_hbm.at[pl.ds(e0, _EDGE_BLK)], uidx)
                pltpu.async_copy(tab_hbm.at[cidx], rows, sem).wait()
                pltpu.sync_copy(rows, acc_sh.at[uidx], add=True)

            plsc.subcore_barrier()
            pltpu.sync_copy(acc_sh.at[pl.ds(r0, stripe)],
                            out_hbm.at[pl.ds(r0, stripe)])
            if tail:
                @pl.when(sid == _N_TILES - 1)
                def _():
                    pltpu.sync_copy(acc_sh.at[pl.ds(stripe * _N_TILES, tail)],
                                    out_hbm.at[pl.ds(stripe * _N_TILES, tail)])

        @pl.when(cid == 0)
        def _():
            run(hl_hbm, zl_hbm)

        @pl.when(cid == 1)
        def _():
            run(hr_hbm, zr_hbm)

    return agg(hL, hR, c2, u2)


def kernel(x, c_2, u_2, batch, We, be, conv_W1, conv_b1, conv_W2, conv_b2,
           Wr1, br1, Wr2, br2):
    num_graphs = 64
    hL, hR = _encoder(x, We, be)
    for i in range(conv_W1.shape[0]):
        zL, zR = _sc_agg(hL, hR, c_2, u_2)
        hL, hR = _mlp(zL, zR, conv_W1[i], conv_b1[i], conv_W2[i], conv_b2[i])
    return _readout(hL, hR, batch, Wr1, br1, Wr2, br2, num_graphs)


# speedup vs baseline: 8.8617x; 2.0843x over previous
import functools

import jax
import jax.numpy as jnp
from jax import lax
from jax.experimental import pallas as pl
from jax.experimental.pallas import tpu as pltpu
from jax.experimental.pallas import tpu_sc as plsc


_DOT = functools.partial(
    lax.dot_general, precision=jax.lax.Precision.HIGHEST,
    preferred_element_type=jnp.float32)


def _mm(a, b):
    return _DOT(a, b, (((a.ndim - 1,), (0,)), ((), ())))


_ROW_BLK = 1000


def _encoder_body(x_ref, we_ref, be_ref, hl_ref, hr_ref):
    h = _mm(x_ref[...], we_ref[...]) + be_ref[...]
    hl_ref[...] = h[:, :128]
    hr_ref[...] = h[:, 128:]


def _encoder(x, We, be):
    n, k = x.shape
    d = We.shape[1]
    grid = (n // _ROW_BLK,)
    return pl.pallas_call(
        _encoder_body,
        grid=grid,
        in_specs=[
            pl.BlockSpec((_ROW_BLK, k), lambda i: (i, 0)),
            pl.BlockSpec((k, d), lambda i: (0, 0)),
            pl.BlockSpec((1, d), lambda i: (0, 0)),
        ],
        out_specs=[
            pl.BlockSpec((_ROW_BLK, d // 2), lambda i: (i, 0)),
            pl.BlockSpec((_ROW_BLK, d // 2), lambda i: (i, 0)),
        ],
        out_shape=[
            jax.ShapeDtypeStruct((n, d // 2), jnp.float32),
            jax.ShapeDtypeStruct((n, d // 2), jnp.float32),
        ],
    )(x, We, be.reshape(1, d))


def _mlp_body(zl_ref, zr_ref, w1t_ref, w1b_ref, b1_ref, w2l_ref, w2r_ref,
              b2l_ref, b2r_ref, hl_ref, hr_ref):
    a = _mm(zl_ref[...], w1t_ref[...]) + _mm(zr_ref[...], w1b_ref[...])
    a = jnp.maximum(a + b1_ref[...], 0.0)
    hl_ref[...] = jnp.maximum(_mm(a, w2l_ref[...]) + b2l_ref[...], 0.0)
    hr_ref[...] = jnp.maximum(_mm(a, w2r_ref[...]) + b2r_ref[...], 0.0)


def _mlp(zL, zR, W1, b1, W2, b2):
    n, dh = zL.shape
    d = 2 * dh
    grid = (n // _ROW_BLK,)
    full = lambda r, c: pl.BlockSpec((r, c), lambda i: (0, 0))
    return pl.pallas_call(
        _mlp_body,
        grid=grid,
        in_specs=[
            pl.BlockSpec((_ROW_BLK, dh), lambda i: (i, 0)),
            pl.BlockSpec((_ROW_BLK, dh), lambda i: (i, 0)),
            full(dh, d), full(dh, d), full(1, d),
            full(d, dh), full(d, dh), full(1, dh), full(1, dh),
        ],
        out_specs=[
            pl.BlockSpec((_ROW_BLK, dh), lambda i: (i, 0)),
            pl.BlockSpec((_ROW_BLK, dh), lambda i: (i, 0)),
        ],
        out_shape=[
            jax.ShapeDtypeStruct((n, dh), jnp.float32),
            jax.ShapeDtypeStruct((n, dh), jnp.float32),
        ],
    )(zL, zR, W1[:dh], W1[dh:], b1.reshape(1, d),
      W2[:, :dh], W2[:, dh:], b2[:dh].reshape(1, dh), b2[dh:].reshape(1, dh))


def _readout_body(hl_ref, hr_ref, b_ref, w1t_ref, w1b_ref, b1_ref, w2_ref,
                  b2_ref, out_ref, accl, accr):
    i = pl.program_id(0)
    ng = accl.shape[0]

    @pl.when(i == 0)
    def _():
        accl[...] = jnp.zeros_like(accl)
        accr[...] = jnp.zeros_like(accr)

    gids = b_ref[0, 0, :]
    onehot = (lax.broadcasted_iota(jnp.int32, (ng, gids.shape[0]), 0)
              == gids[None, :]).astype(jnp.float32)
    accl[...] += _mm(onehot, hl_ref[...])
    accr[...] += _mm(onehot, hr_ref[...])

    @pl.when(i == pl.num_programs(0) - 1)
    def _():
        a = _mm(accl[...], w1t_ref[...]) + _mm(accr[...], w1b_ref[...])
        a = jnp.maximum(a + b1_ref[...], 0.0)
        out_ref[...] = _mm(a, w2_ref[...]) + b2_ref[...]


def _readout(hL, hR, batch, Wr1, br1, Wr2, br2, num_graphs):
    n, dh = hL.shape
    d = 2 * dh
    nc = Wr2.shape[1]
    grid = (n // _ROW_BLK,)
    b3 = batch.reshape(n // _ROW_BLK, 1, _ROW_BLK)
    full = lambda r, c: pl.BlockSpec((r, c), lambda i: (0, 0))
    return pl.pallas_call(
        _readout_body,
        grid=grid,
        in_specs=[
            pl.BlockSpec((_ROW_BLK, dh), lambda i: (i, 0)),
            pl.BlockSpec((_ROW_BLK, dh), lambda i: (i, 0)),
            pl.BlockSpec((1, 1, _ROW_BLK), lambda i: (i, 0, 0)),
            full(dh, d), full(dh, d), full(1, d),
            full(d, nc), full(1, nc),
        ],
        out_specs=pl.BlockSpec((num_graphs, nc), lambda i: (0, 0)),
        out_shape=jax.ShapeDtypeStruct((num_graphs, nc), jnp.float32),
        scratch_shapes=[
            pltpu.VMEM((num_graphs, dh), jnp.float32),
            pltpu.VMEM((num_graphs, dh), jnp.float32),
        ],
    )(hL, hR, b3, Wr1[:dh], Wr1[dh:], br1.reshape(1, d), Wr2,
      br2.reshape(1, nc))


_EDGE_BLK = 80
_N_TILES = 16


def _sc_agg(hL, hR, c2, u2):
    n, dh = hL.shape
    e = c2.shape[0]
    n_blocks = e // _EDGE_BLK
    stripe = (n // _N_TILES) // 8 * 8
    tail = n - stripe * _N_TILES
    mesh = plsc.VectorSubcoreMesh(core_axis_name="c", subcore_axis_name="s")

    nk = n_blocks // _N_TILES
    nk -= nk % 4
    n_extra = n_blocks - nk * _N_TILES

    @functools.partial(
        pl.kernel,
        out_type=(jax.ShapeDtypeStruct((n, dh), jnp.float32),
                  jax.ShapeDtypeStruct((n, dh), jnp.float32)),
        mesh=mesh,
        scratch_types=[
            pltpu.VMEM_SHARED((n, dh), jnp.float32),
            *[pltpu.VMEM((_EDGE_BLK,), jnp.int32) for _ in range(4)],
            *[pltpu.VMEM((_EDGE_BLK,), jnp.int32) for _ in range(4)],
            *[pltpu.VMEM((_EDGE_BLK, dh), jnp.float32) for _ in range(4)],
            *[pltpu.SemaphoreType.DMA for _ in range(12)],
        ],
    )
    def agg(hl_hbm, hr_hbm, c2_hbm, u2_hbm, zl_hbm, zr_hbm, acc_sh,
            ci0, ci1, ci2, ci3, ui0, ui1, ui2, ui3, rw0, rw1, rw2, rw3,
            is0, is1, is2, is3, gs0, gs1, gs2, gs3, ss0, ss1, ss2, ss3):
        cidx = (ci0, ci1, ci2, ci3)
        uidx = (ui0, ui1, ui2, ui3)
        rows = (rw0, rw1, rw2, rw3)
        isem = (is0, is1, is2, is3)
        gsem = (gs0, gs1, gs2, gs3)
        ssem = (ss0, ss1, ss2, ss3)
        cid = lax.axis_index("c")
        sid = lax.axis_index("s")

        def run(tab_hbm, out_hbm):
            r0 = pl.multiple_of(sid * stripe, 8)
            pltpu.sync_copy(tab_hbm.at[pl.ds(r0, stripe)],
                            acc_sh.at[pl.ds(r0, stripe)])
            if tail:
                @pl.when(sid == _N_TILES - 1)
                def _():
                    pltpu.sync_copy(tab_hbm.at[pl.ds(stripe * _N_TILES, tail)],
                                    acc_sh.at[pl.ds(stripe * _N_TILES, tail)])
            plsc.subcore_barrier()

            def e_of(j):
                return (sid + j * _N_TILES) * _EDGE_BLK

            def start_idx(j, s):
                e0 = e_of(j)
                pltpu.async_copy(c2_hbm.at[pl.ds(e0, _EDGE_BLK)],
                                 cidx[s], isem[s])
                pltpu.async_copy(u2_hbm.at[pl.ds(e0, _EDGE_BLK)],
                                 uidx[s], isem[s])

            def wait_idx(s):
                pltpu.make_async_copy(c2_hbm.at[pl.ds(0, _EDGE_BLK)],
                                      cidx[s], isem[s]).wait()
                pltpu.make_async_copy(u2_hbm.at[pl.ds(0, _EDGE_BLK)],
                                      uidx[s], isem[s]).wait()

            def wait_gather(s):
                pltpu.make_async_copy(tab_hbm.at[cidx[s]], rows[s],
                                      gsem[s]).wait()

            def wait_scatter(s):
                pltpu.make_async_copy(rows[s], acc_sh.at[uidx[s]],
                                      ssem[s]).wait()

            start_idx(0, 0)
            start_idx(1, 1)
            wait_idx(0)
            pltpu.async_copy(tab_hbm.at[cidx[0]], rows[0], gsem[0])

            @pl.loop(0, nk, step=4)
            def _(jl):
                for p in range(4):
                    j = jl + p

                    @pl.when(j >= 2)
                    def _():
                        wait_scatter((p + 2) % 4)

                    @pl.when(j + 2 < nk)
                    def _():
                        start_idx(j + 2, (p + 2) % 4)

                    @pl.when(j + 1 < nk)
                    def _():
                        wait_idx((p + 1) % 4)
                        pltpu.async_copy(tab_hbm.at[cidx[(p + 1) % 4]],
                                         rows[(p + 1) % 4], gsem[(p + 1) % 4])

                    wait_gather(p)
                    pltpu.async_copy(rows[p], acc_sh.at[uidx[p]], ssem[p],
                                     add=True)

            wait_scatter((nk - 2) % 4)
            wait_scatter((nk - 1) % 4)

            for x in range(n_extra // _N_TILES):
                e0 = (nk * _N_TILES + x * _N_TILES + sid) * _EDGE_BLK
                pltpu.sync_copy(c2_hbm.at[pl.ds(e0, _EDGE_BLK)], ci0)
                pltpu.sync_copy(u2_hbm.at[pl.ds(e0, _EDGE_BLK)], ui0)
                pltpu.sync_copy(tab_hbm.at[ci0], rw0)
                pltpu.sync_copy(rw0, acc_sh.at[ui0], add=True)
            rem = n_extra % _N_TILES
            if rem:
                @pl.when(sid < rem)
                def _():
                    e0 = ((n_blocks - rem) + sid) * _EDGE_BLK
                    pltpu.sync_copy(c2_hbm.at[pl.ds(e0, _EDGE_BLK)], ci0)
                    pltpu.sync_copy(u2_hbm.at[pl.ds(e0, _EDGE_BLK)], ui0)
                    pltpu.sync_copy(tab_hbm.at[ci0], rw0)
                    pltpu.sync_copy(rw0, acc_sh.at[ui0], add=True)

            plsc.subcore_barrier()
            pltpu.sync_copy(acc_sh.at[pl.ds(r0, stripe)],
                            out_hbm.at[pl.ds(r0, stripe)])
            if tail:
                @pl.when(sid == _N_TILES - 1)
                def _():
                    pltpu.sync_copy(acc_sh.at[pl.ds(stripe * _N_TILES, tail)],
                                    out_hbm.at[pl.ds(stripe * _N_TILES, tail)])

        @pl.when(cid == 0)
        def _():
            run(hl_hbm, zl_hbm)

        @pl.when(cid == 1)
        def _():
            run(hr_hbm, zr_hbm)

    return agg(hL, hR, c2, u2)


def kernel(x, c_2, u_2, batch, We, be, conv_W1, conv_b1, conv_W2, conv_b2,
           Wr1, br1, Wr2, br2):
    num_graphs = 64
    hL, hR = _encoder(x, We, be)
    for i in range(conv_W1.shape[0]):
        zL, zR = _sc_agg(hL, hR, c_2, u_2)
        hL, hR = _mlp(zL, zR, conv_W1[i], conv_b1[i], conv_W2[i], conv_b2[i])
    return _readout(hL, hR, batch, Wr1, br1, Wr2, br2, num_graphs)


# speedup vs baseline: 8.9057x; 1.0050x over previous
import functools

import jax
import jax.numpy as jnp
from jax import lax
from jax.experimental import pallas as pl
from jax.experimental.pallas import tpu as pltpu
from jax.experimental.pallas import tpu_sc as plsc


_DOT = functools.partial(
    lax.dot_general, precision=jax.lax.Precision.HIGHEST,
    preferred_element_type=jnp.float32)


def _mm(a, b):
    return _DOT(a, b, (((a.ndim - 1,), (0,)), ((), ())))


_ROW_BLK = 1000


def _encoder_body(x_ref, we_ref, be_ref, hl_ref, hr_ref):
    h = _mm(x_ref[...], we_ref[...]) + be_ref[...]
    hl_ref[...] = h[:, :128]
    hr_ref[...] = h[:, 128:]


def _encoder(x, We, be):
    n, k = x.shape
    d = We.shape[1]
    grid = (n // _ROW_BLK,)
    return pl.pallas_call(
        _encoder_body,
        grid=grid,
        in_specs=[
            pl.BlockSpec((_ROW_BLK, k), lambda i: (i, 0)),
            pl.BlockSpec((k, d), lambda i: (0, 0)),
            pl.BlockSpec((1, d), lambda i: (0, 0)),
        ],
        out_specs=[
            pl.BlockSpec((_ROW_BLK, d // 2), lambda i: (i, 0)),
            pl.BlockSpec((_ROW_BLK, d // 2), lambda i: (i, 0)),
        ],
        out_shape=[
            jax.ShapeDtypeStruct((n, d // 2), jnp.float32),
            jax.ShapeDtypeStruct((n, d // 2), jnp.float32),
        ],
    )(x, We, be.reshape(1, d))


def _mlp_body(zl_ref, zr_ref, w1t_ref, w1b_ref, b1_ref, w2l_ref, w2r_ref,
              b2l_ref, b2r_ref, hl_ref, hr_ref):
    a = _mm(zl_ref[...], w1t_ref[...]) + _mm(zr_ref[...], w1b_ref[...])
    a = jnp.maximum(a + b1_ref[...], 0.0)
    hl_ref[...] = jnp.maximum(_mm(a, w2l_ref[...]) + b2l_ref[...], 0.0)
    hr_ref[...] = jnp.maximum(_mm(a, w2r_ref[...]) + b2r_ref[...], 0.0)


def _mlp(zL, zR, W1, b1, W2, b2):
    n, dh = zL.shape
    d = 2 * dh
    grid = (n // _ROW_BLK,)
    full = lambda r, c: pl.BlockSpec((r, c), lambda i: (0, 0))
    return pl.pallas_call(
        _mlp_body,
        grid=grid,
        in_specs=[
            pl.BlockSpec((_ROW_BLK, dh), lambda i: (i, 0)),
            pl.BlockSpec((_ROW_BLK, dh), lambda i: (i, 0)),
            full(dh, d), full(dh, d), full(1, d),
            full(d, dh), full(d, dh), full(1, dh), full(1, dh),
        ],
        out_specs=[
            pl.BlockSpec((_ROW_BLK, dh), lambda i: (i, 0)),
            pl.BlockSpec((_ROW_BLK, dh), lambda i: (i, 0)),
        ],
        out_shape=[
            jax.ShapeDtypeStruct((n, dh), jnp.float32),
            jax.ShapeDtypeStruct((n, dh), jnp.float32),
        ],
    )(zL, zR, W1[:dh], W1[dh:], b1.reshape(1, d),
      W2[:, :dh], W2[:, dh:], b2[:dh].reshape(1, dh), b2[dh:].reshape(1, dh))


def _readout_body(hl_ref, hr_ref, b_ref, w1t_ref, w1b_ref, b1_ref, w2_ref,
                  b2_ref, out_ref, accl, accr):
    i = pl.program_id(0)
    ng = accl.shape[0]

    @pl.when(i == 0)
    def _():
        accl[...] = jnp.zeros_like(accl)
        accr[...] = jnp.zeros_like(accr)

    gids = b_ref[0, 0, :]
    onehot = (lax.broadcasted_iota(jnp.int32, (ng, gids.shape[0]), 0)
              == gids[None, :]).astype(jnp.float32)
    accl[...] += _mm(onehot, hl_ref[...])
    accr[...] += _mm(onehot, hr_ref[...])

    @pl.when(i == pl.num_programs(0) - 1)
    def _():
        a = _mm(accl[...], w1t_ref[...]) + _mm(accr[...], w1b_ref[...])
        a = jnp.maximum(a + b1_ref[...], 0.0)
        out_ref[...] = _mm(a, w2_ref[...]) + b2_ref[...]


def _readout(hL, hR, batch, Wr1, br1, Wr2, br2, num_graphs):
    n, dh = hL.shape
    d = 2 * dh
    nc = Wr2.shape[1]
    grid = (n // _ROW_BLK,)
    b3 = batch.reshape(n // _ROW_BLK, 1, _ROW_BLK)
    full = lambda r, c: pl.BlockSpec((r, c), lambda i: (0, 0))
    return pl.pallas_call(
        _readout_body,
        grid=grid,
        in_specs=[
            pl.BlockSpec((_ROW_BLK, dh), lambda i: (i, 0)),
            pl.BlockSpec((_ROW_BLK, dh), lambda i: (i, 0)),
            pl.BlockSpec((1, 1, _ROW_BLK), lambda i: (i, 0, 0)),
            full(dh, d), full(dh, d), full(1, d),
            full(d, nc), full(1, nc),
        ],
        out_specs=pl.BlockSpec((num_graphs, nc), lambda i: (0, 0)),
        out_shape=jax.ShapeDtypeStruct((num_graphs, nc), jnp.float32),
        scratch_shapes=[
            pltpu.VMEM((num_graphs, dh), jnp.float32),
            pltpu.VMEM((num_graphs, dh), jnp.float32),
        ],
    )(hL, hR, b3, Wr1[:dh], Wr1[dh:], br1.reshape(1, d), Wr2,
      br2.reshape(1, nc))


_EDGE_BLK = 80
_N_TILES = 16


def _sc_agg(hL, hR, c2, u2):
    n, dh = hL.shape
    e = c2.shape[0]
    n_blocks = e // _EDGE_BLK
    stripe = (n // _N_TILES) // 8 * 8
    tail = n - stripe * _N_TILES
    mesh = plsc.VectorSubcoreMesh(core_axis_name="c", subcore_axis_name="s")

    nk = n_blocks // _N_TILES
    assert nk * _N_TILES == n_blocks
    _RR = 4
    _IR = 5
    _UNROLL = 20

    @functools.partial(
        pl.kernel,
        out_type=(jax.ShapeDtypeStruct((n, dh), jnp.float32),
                  jax.ShapeDtypeStruct((n, dh), jnp.float32)),
        mesh=mesh,
        scratch_types=[
            pltpu.VMEM_SHARED((n, dh), jnp.float32),
            *[pltpu.VMEM((_EDGE_BLK,), jnp.int32) for _ in range(2 * _IR)],
            *[pltpu.VMEM((_EDGE_BLK, dh), jnp.float32) for _ in range(_RR)],
            *[pltpu.SemaphoreType.DMA for _ in range(_IR + 2 * _RR)],
        ],
    )
    def agg(hl_hbm, hr_hbm, c2_hbm, u2_hbm, zl_hbm, zr_hbm, acc_sh,
            ci0, ci1, ci2, ci3, ci4, ui0, ui1, ui2, ui3, ui4,
            rw0, rw1, rw2, rw3,
            is0, is1, is2, is3, is4, gs0, gs1, gs2, gs3, ss0, ss1, ss2, ss3):
        cidx = (ci0, ci1, ci2, ci3, ci4)
        uidx = (ui0, ui1, ui2, ui3, ui4)
        rows = (rw0, rw1, rw2, rw3)
        isem = (is0, is1, is2, is3, is4)
        gsem = (gs0, gs1, gs2, gs3)
        ssem = (ss0, ss1, ss2, ss3)
        cid = lax.axis_index("c")
        sid = lax.axis_index("s")

        def run(tab_hbm, out_hbm):
            r0 = pl.multiple_of(sid * stripe, 8)
            pltpu.sync_copy(tab_hbm.at[pl.ds(r0, stripe)],
                            acc_sh.at[pl.ds(r0, stripe)])
            if tail:
                @pl.when(sid == _N_TILES - 1)
                def _():
                    pltpu.sync_copy(tab_hbm.at[pl.ds(stripe * _N_TILES, tail)],
                                    acc_sh.at[pl.ds(stripe * _N_TILES, tail)])
            plsc.subcore_barrier()

            def e_of(j):
                return (sid + j * _N_TILES) * _EDGE_BLK

            def start_idx(j, s):
                e0 = e_of(j)
                pltpu.async_copy(c2_hbm.at[pl.ds(e0, _EDGE_BLK)],
                                 cidx[s], isem[s])
                pltpu.async_copy(u2_hbm.at[pl.ds(e0, _EDGE_BLK)],
                                 uidx[s], isem[s])

            def wait_idx(s):
                pltpu.make_async_copy(c2_hbm.at[pl.ds(0, _EDGE_BLK)],
                                      cidx[s], isem[s]).wait()
                pltpu.make_async_copy(u2_hbm.at[pl.ds(0, _EDGE_BLK)],
                                      uidx[s], isem[s]).wait()

            def wait_gather(rs, ixs):
                pltpu.make_async_copy(tab_hbm.at[cidx[ixs]], rows[rs],
                                      gsem[rs]).wait()

            def wait_scatter(rs, ixs):
                pltpu.make_async_copy(rows[rs], acc_sh.at[uidx[ixs]],
                                      ssem[rs]).wait()

            def stage(j, p4, p5, a1_pred, do_a2, do_b):
                def a1():
                    wait_scatter((p4 + 2) % _RR, (p5 + 3) % _IR)
                if a1_pred is True:
                    a1()
                elif a1_pred is not False:
                    pl.when(a1_pred)(a1)
                if do_a2:
                    start_idx(j + 3, (p5 + 3) % _IR)
                if do_b:
                    wait_idx((p5 + 2) % _IR)
                    pltpu.async_copy(tab_hbm.at[cidx[(p5 + 2) % _IR]],
                                     rows[(p4 + 2) % _RR],
                                     gsem[(p4 + 2) % _RR])
                wait_gather(p4, p5)
                pltpu.async_copy(rows[p4], acc_sh.at[uidx[p5]], ssem[p4],
                                 add=True)

            for j in range(3):
                start_idx(j, j)
            for j in range(2):
                wait_idx(j)
                pltpu.async_copy(tab_hbm.at[cidx[j]], rows[j], gsem[j])

            main = (nk - _UNROLL // 2) // _UNROLL * _UNROLL

            @pl.loop(0, main, step=_UNROLL)
            def _(jl):
                for p in range(_UNROLL):
                    j = jl + p
                    pred = True if p >= 2 else (j >= 2)
                    stage(j, p % _RR, p % _IR, pred, True, True)

            for j in range(main, nk):
                stage(j, j % _RR, j % _IR, True, j + 3 < nk, j + 2 < nk)

            wait_scatter((nk - 2) % _RR, (nk - 2) % _IR)
            wait_scatter((nk - 1) % _RR, (nk - 1) % _IR)

            plsc.subcore_barrier()
            pltpu.sync_copy(acc_sh.at[pl.ds(r0, stripe)],
                            out_hbm.at[pl.ds(r0, stripe)])
            if tail:
                @pl.when(sid == _N_TILES - 1)
                def _():
                    pltpu.sync_copy(acc_sh.at[pl.ds(stripe * _N_TILES, tail)],
                                    out_hbm.at[pl.ds(stripe * _N_TILES, tail)])

        @pl.when(cid == 0)
        def _():
            run(hl_hbm, zl_hbm)

        @pl.when(cid == 1)
        def _():
            run(hr_hbm, zr_hbm)

    return agg(hL, hR, c2, u2)


def kernel(x, c_2, u_2, batch, We, be, conv_W1, conv_b1, conv_W2, conv_b2,
           Wr1, br1, Wr2, br2):
    num_graphs = 64
    hL, hR = _encoder(x, We, be)
    for i in range(conv_W1.shape[0]):
        zL, zR = _sc_agg(hL, hR, c_2, u_2)
        hL, hR = _mlp(zL, zR, conv_W1[i], conv_b1[i], conv_W2[i], conv_b2[i])
    return _readout(hL, hR, batch, Wr1, br1, Wr2, br2, num_graphs)


# speedup vs baseline: 11.2478x; 1.2630x over previous
import functools

import jax
import jax.numpy as jnp
from jax import lax
from jax.experimental import pallas as pl
from jax.experimental.pallas import tpu as pltpu
from jax.experimental.pallas import tpu_sc as plsc


_DOT = functools.partial(
    lax.dot_general, precision=jax.lax.Precision.DEFAULT,
    preferred_element_type=jnp.float32)


def _mm(a, b):
    return _DOT(a, b, (((a.ndim - 1,), (0,)), ((), ())))


_ROW_BLK = 1000


def _encoder_body(x_ref, we_ref, be_ref, hl_ref, hr_ref):
    h = _mm(x_ref[...], we_ref[...]) + be_ref[...]
    hl_ref[...] = h[:, :128]
    hr_ref[...] = h[:, 128:]


def _encoder(x, We, be):
    n, k = x.shape
    d = We.shape[1]
    grid = (n // _ROW_BLK,)
    return pl.pallas_call(
        _encoder_body,
        grid=grid,
        in_specs=[
            pl.BlockSpec((_ROW_BLK, k), lambda i: (i, 0)),
            pl.BlockSpec((k, d), lambda i: (0, 0)),
            pl.BlockSpec((1, d), lambda i: (0, 0)),
        ],
        out_specs=[
            pl.BlockSpec((_ROW_BLK, d // 2), lambda i: (i, 0)),
            pl.BlockSpec((_ROW_BLK, d // 2), lambda i: (i, 0)),
        ],
        out_shape=[
            jax.ShapeDtypeStruct((n, d // 2), jnp.float32),
            jax.ShapeDtypeStruct((n, d // 2), jnp.float32),
        ],
    )(x, We, be.reshape(1, d))


def _mlp_body(zl_ref, zr_ref, w1t_ref, w1b_ref, b1_ref, w2l_ref, w2r_ref,
              b2l_ref, b2r_ref, hl_ref, hr_ref):
    a = _mm(zl_ref[...], w1t_ref[...]) + _mm(zr_ref[...], w1b_ref[...])
    a = jnp.maximum(a + b1_ref[...], 0.0)
    hl_ref[...] = jnp.maximum(_mm(a, w2l_ref[...]) + b2l_ref[...], 0.0)
    hr_ref[...] = jnp.maximum(_mm(a, w2r_ref[...]) + b2r_ref[...], 0.0)


def _mlp(zL, zR, W1, b1, W2, b2):
    n, dh = zL.shape
    d = 2 * dh
    grid = (n // _ROW_BLK,)
    full = lambda r, c: pl.BlockSpec((r, c), lambda i: (0, 0))
    return pl.pallas_call(
        _mlp_body,
        grid=grid,
        in_specs=[
            pl.BlockSpec((_ROW_BLK, dh), lambda i: (i, 0)),
            pl.BlockSpec((_ROW_BLK, dh), lambda i: (i, 0)),
            full(dh, d), full(dh, d), full(1, d),
            full(d, dh), full(d, dh), full(1, dh), full(1, dh),
        ],
        out_specs=[
            pl.BlockSpec((_ROW_BLK, dh), lambda i: (i, 0)),
            pl.BlockSpec((_ROW_BLK, dh), lambda i: (i, 0)),
        ],
        out_shape=[
            jax.ShapeDtypeStruct((n, dh), jnp.float32),
            jax.ShapeDtypeStruct((n, dh), jnp.float32),
        ],
    )(zL, zR, W1[:dh], W1[dh:], b1.reshape(1, d),
      W2[:, :dh], W2[:, dh:], b2[:dh].reshape(1, dh), b2[dh:].reshape(1, dh))


def _readout_body(hl_ref, hr_ref, b_ref, w1t_ref, w1b_ref, b1_ref, w2_ref,
                  b2_ref, out_ref, accl, accr):
    i = pl.program_id(0)
    ng = accl.shape[0]

    @pl.when(i == 0)
    def _():
        accl[...] = jnp.zeros_like(accl)
        accr[...] = jnp.zeros_like(accr)

    gids = b_ref[0, 0, :]
    onehot = (lax.broadcasted_iota(jnp.int32, (ng, gids.shape[0]), 0)
              == gids[None, :]).astype(jnp.float32)
    accl[...] += _mm(onehot, hl_ref[...])
    accr[...] += _mm(onehot, hr_ref[...])

    @pl.when(i == pl.num_programs(0) - 1)
    def _():
        a = _mm(accl[...], w1t_ref[...]) + _mm(accr[...], w1b_ref[...])
        a = jnp.maximum(a + b1_ref[...], 0.0)
        out_ref[...] = _mm(a, w2_ref[...]) + b2_ref[...]


def _readout(hL, hR, batch, Wr1, br1, Wr2, br2, num_graphs):
    n, dh = hL.shape
    d = 2 * dh
    nc = Wr2.shape[1]
    grid = (n // _ROW_BLK,)
    b3 = batch.reshape(n // _ROW_BLK, 1, _ROW_BLK)
    full = lambda r, c: pl.BlockSpec((r, c), lambda i: (0, 0))
    return pl.pallas_call(
        _readout_body,
        grid=grid,
        in_specs=[
            pl.BlockSpec((_ROW_BLK, dh), lambda i: (i, 0)),
            pl.BlockSpec((_ROW_BLK, dh), lambda i: (i, 0)),
            pl.BlockSpec((1, 1, _ROW_BLK), lambda i: (i, 0, 0)),
            full(dh, d), full(dh, d), full(1, d),
            full(d, nc), full(1, nc),
        ],
        out_specs=pl.BlockSpec((num_graphs, nc), lambda i: (0, 0)),
        out_shape=jax.ShapeDtypeStruct((num_graphs, nc), jnp.float32),
        scratch_shapes=[
            pltpu.VMEM((num_graphs, dh), jnp.float32),
            pltpu.VMEM((num_graphs, dh), jnp.float32),
        ],
    )(hL, hR, b3, Wr1[:dh], Wr1[dh:], br1.reshape(1, d), Wr2,
      br2.reshape(1, nc))


_EDGE_BLK = 80
_N_TILES = 16


def _sc_agg(hL, hR, c2, u2):
    n, dh = hL.shape
    e = c2.shape[0]
    n_blocks = e // _EDGE_BLK
    stripe = (n // _N_TILES) // 8 * 8
    tail = n - stripe * _N_TILES
    mesh = plsc.VectorSubcoreMesh(core_axis_name="c", subcore_axis_name="s")

    nk = n_blocks // _N_TILES
    assert nk * _N_TILES == n_blocks
    _RR = 4
    _IR = 5
    _UNROLL = 20

    @functools.partial(
        pl.kernel,
        out_type=(jax.ShapeDtypeStruct((n, dh), jnp.float32),
                  jax.ShapeDtypeStruct((n, dh), jnp.float32)),
        mesh=mesh,
        scratch_types=[
            pltpu.VMEM_SHARED((n, dh), jnp.float32),
            *[pltpu.VMEM((_EDGE_BLK,), jnp.int32) for _ in range(2 * _IR)],
            *[pltpu.VMEM((_EDGE_BLK, dh), jnp.float32) for _ in range(_RR)],
            *[pltpu.SemaphoreType.DMA for _ in range(_IR + 2 * _RR)],
        ],
    )
    def agg(hl_hbm, hr_hbm, c2_hbm, u2_hbm, zl_hbm, zr_hbm, acc_sh,
            ci0, ci1, ci2, ci3, ci4, ui0, ui1, ui2, ui3, ui4,
            rw0, rw1, rw2, rw3,
            is0, is1, is2, is3, is4, gs0, gs1, gs2, gs3, ss0, ss1, ss2, ss3):
        cidx = (ci0, ci1, ci2, ci3, ci4)
        uidx = (ui0, ui1, ui2, ui3, ui4)
        rows = (rw0, rw1, rw2, rw3)
        isem = (is0, is1, is2, is3, is4)
        gsem = (gs0, gs1, gs2, gs3)
        ssem = (ss0, ss1, ss2, ss3)
        cid = lax.axis_index("c")
        sid = lax.axis_index("s")

        def run(tab_hbm, out_hbm):
            r0 = pl.multiple_of(sid * stripe, 8)
            pltpu.sync_copy(tab_hbm.at[pl.ds(r0, stripe)],
                            acc_sh.at[pl.ds(r0, stripe)])
            if tail:
                @pl.when(sid == _N_TILES - 1)
                def _():
                    pltpu.sync_copy(tab_hbm.at[pl.ds(stripe * _N_TILES, tail)],
                                    acc_sh.at[pl.ds(stripe * _N_TILES, tail)])
            plsc.subcore_barrier()

            def e_of(j):
                return (sid + j * _N_TILES) * _EDGE_BLK

            def start_idx(j, s):
                e0 = e_of(j)
                pltpu.async_copy(c2_hbm.at[pl.ds(e0, _EDGE_BLK)],
                                 cidx[s], isem[s])
                pltpu.async_copy(u2_hbm.at[pl.ds(e0, _EDGE_BLK)],
                                 uidx[s], isem[s])

            def wait_idx(s):
                pltpu.make_async_copy(c2_hbm.at[pl.ds(0, _EDGE_BLK)],
                                      cidx[s], isem[s]).wait()
                pltpu.make_async_copy(u2_hbm.at[pl.ds(0, _EDGE_BLK)],
                                      uidx[s], isem[s]).wait()

            def wait_gather(rs, ixs):
                pltpu.make_async_copy(tab_hbm.at[cidx[ixs]], rows[rs],
                                      gsem[rs]).wait()

            def wait_scatter(rs, ixs):
                pltpu.make_async_copy(rows[rs], acc_sh.at[uidx[ixs]],
                                      ssem[rs]).wait()

            def stage(j, p4, p5, a1_pred, do_a2, do_b):
                def a1():
                    wait_scatter((p4 + 2) % _RR, (p5 + 3) % _IR)
                if a1_pred is True:
                    a1()
                elif a1_pred is not False:
                    pl.when(a1_pred)(a1)
                if do_a2:
                    start_idx(j + 3, (p5 + 3) % _IR)
                if do_b:
                    wait_idx((p5 + 2) % _IR)
                    pltpu.async_copy(tab_hbm.at[cidx[(p5 + 2) % _IR]],
                                     rows[(p4 + 2) % _RR],
                                     gsem[(p4 + 2) % _RR])
                wait_gather(p4, p5)
                pltpu.async_copy(rows[p4], acc_sh.at[uidx[p5]], ssem[p4],
                                 add=True)

            for j in range(3):
                start_idx(j, j)
            for j in range(2):
                wait_idx(j)
                pltpu.async_copy(tab_hbm.at[cidx[j]], rows[j], gsem[j])

            main = (nk - _UNROLL // 2) // _UNROLL * _UNROLL

            @pl.loop(0, main, step=_UNROLL)
            def _(jl):
                for p in range(_UNROLL):
                    j = jl + p
                    pred = True if p >= 2 else (j >= 2)
                    stage(j, p % _RR, p % _IR, pred, True, True)

            for j in range(main, nk):
                stage(j, j % _RR, j % _IR, True, j + 3 < nk, j + 2 < nk)

            wait_scatter((nk - 2) % _RR, (nk - 2) % _IR)
            wait_scatter((nk - 1) % _RR, (nk - 1) % _IR)

            plsc.subcore_barrier()
            pltpu.sync_copy(acc_sh.at[pl.ds(r0, stripe)],
                            out_hbm.at[pl.ds(r0, stripe)])
            if tail:
                @pl.when(sid == _N_TILES - 1)
                def _():
                    pltpu.sync_copy(acc_sh.at[pl.ds(stripe * _N_TILES, tail)],
                                    out_hbm.at[pl.ds(stripe * _N_TILES, tail)])

        @pl.when(cid == 0)
        def _():
            run(hl_hbm, zl_hbm)

        @pl.when(cid == 1)
        def _():
            run(hr_hbm, zr_hbm)

    return agg(hL, hR, c2, u2)


def kernel(x, c_2, u_2, batch, We, be, conv_W1, conv_b1, conv_W2, conv_b2,
           Wr1, br1, Wr2, br2):
    num_graphs = 64
    hL, hR = _encoder(x, We, be)
    for i in range(conv_W1.shape[0]):
        zL, zR = _sc_agg(hL, hR, c_2, u_2)
        hL, hR = _mlp(zL, zR, conv_W1[i], conv_b1[i], conv_W2[i], conv_b2[i])
    return _readout(hL, hR, batch, Wr1, br1, Wr2, br2, num_graphs)


# speedup vs baseline: 11.3743x; 1.0113x over previous
import functools

import jax
import jax.numpy as jnp
from jax import lax
from jax.experimental import pallas as pl
from jax.experimental.pallas import tpu as pltpu
from jax.experimental.pallas import tpu_sc as plsc


_DOT = functools.partial(
    lax.dot_general, precision=jax.lax.Precision.DEFAULT,
    preferred_element_type=jnp.float32)


def _mm(a, b):
    return _DOT(a, b, (((a.ndim - 1,), (0,)), ((), ())))


_ROW_BLK = 1000


def _encoder_body(x_ref, we_ref, be_ref, hl_ref, hr_ref):
    h = _mm(x_ref[...], we_ref[...]) + be_ref[...]
    hl_ref[...] = h[:, :128]
    hr_ref[...] = h[:, 128:]


def _encoder(x, We, be):
    n, k = x.shape
    d = We.shape[1]
    grid = (n // _ROW_BLK,)
    return pl.pallas_call(
        _encoder_body,
        grid=grid,
        in_specs=[
            pl.BlockSpec((_ROW_BLK, k), lambda i: (i, 0)),
            pl.BlockSpec((k, d), lambda i: (0, 0)),
            pl.BlockSpec((1, d), lambda i: (0, 0)),
        ],
        out_specs=[
            pl.BlockSpec((_ROW_BLK, d // 2), lambda i: (i, 0)),
            pl.BlockSpec((_ROW_BLK, d // 2), lambda i: (i, 0)),
        ],
        out_shape=[
            jax.ShapeDtypeStruct((n, d // 2), jnp.float32),
            jax.ShapeDtypeStruct((n, d // 2), jnp.float32),
        ],
    )(x, We, be.reshape(1, d))


def _mlp_body(zl_ref, zr_ref, w1t_ref, w1b_ref, b1_ref, w2l_ref, w2r_ref,
              b2l_ref, b2r_ref, hl_ref, hr_ref):
    a = _mm(zl_ref[...], w1t_ref[...]) + _mm(zr_ref[...], w1b_ref[...])
    a = jnp.maximum(a + b1_ref[...], 0.0)
    hl_ref[...] = jnp.maximum(_mm(a, w2l_ref[...]) + b2l_ref[...], 0.0)
    hr_ref[...] = jnp.maximum(_mm(a, w2r_ref[...]) + b2r_ref[...], 0.0)


def _mlp(zL, zR, W1, b1, W2, b2):
    n, dh = zL.shape
    d = 2 * dh
    grid = (n // _ROW_BLK,)
    full = lambda r, c: pl.BlockSpec((r, c), lambda i: (0, 0))
    return pl.pallas_call(
        _mlp_body,
        grid=grid,
        in_specs=[
            pl.BlockSpec((_ROW_BLK, dh), lambda i: (i, 0)),
            pl.BlockSpec((_ROW_BLK, dh), lambda i: (i, 0)),
            full(dh, d), full(dh, d), full(1, d),
            full(d, dh), full(d, dh), full(1, dh), full(1, dh),
        ],
        out_specs=[
            pl.BlockSpec((_ROW_BLK, dh), lambda i: (i, 0)),
            pl.BlockSpec((_ROW_BLK, dh), lambda i: (i, 0)),
        ],
        out_shape=[
            jax.ShapeDtypeStruct((n, dh), jnp.float32),
            jax.ShapeDtypeStruct((n, dh), jnp.float32),
        ],
    )(zL, zR, W1[:dh], W1[dh:], b1.reshape(1, d),
      W2[:, :dh], W2[:, dh:], b2[:dh].reshape(1, dh), b2[dh:].reshape(1, dh))


def _readout_body(hl_ref, hr_ref, b_ref, w1t_ref, w1b_ref, b1_ref, w2_ref,
                  b2_ref, out_ref, accl, accr):
    i = pl.program_id(0)
    ng = accl.shape[0]

    @pl.when(i == 0)
    def _():
        accl[...] = jnp.zeros_like(accl)
        accr[...] = jnp.zeros_like(accr)

    gids = b_ref[0, 0, :]
    onehot = (lax.broadcasted_iota(jnp.int32, (ng, gids.shape[0]), 0)
              == gids[None, :]).astype(jnp.float32)
    accl[...] += _mm(onehot, hl_ref[...])
    accr[...] += _mm(onehot, hr_ref[...])

    @pl.when(i == pl.num_programs(0) - 1)
    def _():
        a = _mm(accl[...], w1t_ref[...]) + _mm(accr[...], w1b_ref[...])
        a = jnp.maximum(a + b1_ref[...], 0.0)
        out_ref[...] = _mm(a, w2_ref[...]) + b2_ref[...]


def _readout(hL, hR, batch, Wr1, br1, Wr2, br2, num_graphs):
    n, dh = hL.shape
    d = 2 * dh
    nc = Wr2.shape[1]
    grid = (n // _ROW_BLK,)
    b3 = batch.reshape(n // _ROW_BLK, 1, _ROW_BLK)
    full = lambda r, c: pl.BlockSpec((r, c), lambda i: (0, 0))
    return pl.pallas_call(
        _readout_body,
        grid=grid,
        in_specs=[
            pl.BlockSpec((_ROW_BLK, dh), lambda i: (i, 0)),
            pl.BlockSpec((_ROW_BLK, dh), lambda i: (i, 0)),
            pl.BlockSpec((1, 1, _ROW_BLK), lambda i: (i, 0, 0)),
            full(dh, d), full(dh, d), full(1, d),
            full(d, nc), full(1, nc),
        ],
        out_specs=pl.BlockSpec((num_graphs, nc), lambda i: (0, 0)),
        out_shape=jax.ShapeDtypeStruct((num_graphs, nc), jnp.float32),
        scratch_shapes=[
            pltpu.VMEM((num_graphs, dh), jnp.float32),
            pltpu.VMEM((num_graphs, dh), jnp.float32),
        ],
    )(hL, hR, b3, Wr1[:dh], Wr1[dh:], br1.reshape(1, d), Wr2,
      br2.reshape(1, nc))


_EDGE_BLK = 128
_N_TILES = 16


def _sc_agg(hL, hR, c2, u2):
    n, dh = hL.shape
    e = c2.shape[0]
    n_blocks = e // _EDGE_BLK
    stripe = (n // _N_TILES) // 8 * 8
    tail = n - stripe * _N_TILES
    mesh = plsc.VectorSubcoreMesh(core_axis_name="c", subcore_axis_name="s")

    nk = n_blocks // _N_TILES
    n_extra = n_blocks - nk * _N_TILES
    _RR = 3
    _IR = 4
    _UNROLL = 12
    assert nk % _UNROLL == 0 and n_extra <= _N_TILES

    @functools.partial(
        pl.kernel,
        out_type=(jax.ShapeDtypeStruct((n, dh), jnp.float32),
                  jax.ShapeDtypeStruct((n, dh), jnp.float32)),
        mesh=mesh,
        scratch_types=[
            pltpu.VMEM_SHARED((n, dh), jnp.float32),
            *[pltpu.VMEM((_EDGE_BLK,), jnp.int32) for _ in range(2 * _IR)],
            *[pltpu.VMEM((_EDGE_BLK, dh), jnp.float32) for _ in range(_RR)],
            *[pltpu.SemaphoreType.DMA for _ in range(_IR + 2 * _RR + 1)],
        ],
    )
    def agg(hl_hbm, hr_hbm, c2_hbm, u2_hbm, zl_hbm, zr_hbm, acc_sh,
            ci0, ci1, ci2, ci3, ui0, ui1, ui2, ui3,
            rw0, rw1, rw2,
            is0, is1, is2, is3, gs0, gs1, gs2, ss0, ss1, ss2, stsem):
        cidx = (ci0, ci1, ci2, ci3)
        uidx = (ui0, ui1, ui2, ui3)
        rows = (rw0, rw1, rw2)
        isem = (is0, is1, is2, is3)
        gsem = (gs0, gs1, gs2)
        ssem = (ss0, ss1, ss2)
        cid = lax.axis_index("c")
        sid = lax.axis_index("s")

        def run(tab_hbm, out_hbm):
            r0 = pl.multiple_of(sid * stripe, 8)

            def e_of(j):
                return (sid + j * _N_TILES) * _EDGE_BLK

            def start_idx(j, s):
                e0 = e_of(j)
                pltpu.async_copy(c2_hbm.at[pl.ds(e0, _EDGE_BLK)],
                                 cidx[s], isem[s])
                pltpu.async_copy(u2_hbm.at[pl.ds(e0, _EDGE_BLK)],
                                 uidx[s], isem[s])

            def wait_idx(s):
                pltpu.make_async_copy(c2_hbm.at[pl.ds(0, _EDGE_BLK)],
                                      cidx[s], isem[s]).wait()
                pltpu.make_async_copy(u2_hbm.at[pl.ds(0, _EDGE_BLK)],
                                      uidx[s], isem[s]).wait()

            def wait_gather(rs, ixs):
                pltpu.make_async_copy(tab_hbm.at[cidx[ixs]], rows[rs],
                                      gsem[rs]).wait()

            def wait_scatter(rs, ixs):
                pltpu.make_async_copy(rows[rs], acc_sh.at[uidx[ixs]],
                                      ssem[rs]).wait()

            def stage(j, pr, pi, a1_pred, do_a2, do_b):
                def a1():
                    wait_scatter((pr + 1) % _RR, (pi + 2) % _IR)
                if a1_pred is True:
                    a1()
                elif a1_pred is not False:
                    pl.when(a1_pred)(a1)
                if do_a2:
                    start_idx(j + 2, (pi + 2) % _IR)
                if do_b:
                    wait_idx((pi + 1) % _IR)
                    pltpu.async_copy(tab_hbm.at[cidx[(pi + 1) % _IR]],
                                     rows[(pr + 1) % _RR],
                                     gsem[(pr + 1) % _RR])
                wait_gather(pr, pi)
                pltpu.async_copy(rows[pr], acc_sh.at[uidx[pi]], ssem[pr],
                                 add=True)

            for j in range(2):
                start_idx(j, j)
            wait_idx(0)
            pltpu.async_copy(tab_hbm.at[cidx[0]], rows[0], gsem[0])
            st = pltpu.async_copy(tab_hbm.at[pl.ds(r0, stripe)],
                                  acc_sh.at[pl.ds(r0, stripe)], stsem)
            if tail:
                @pl.when(sid == _N_TILES - 1)
                def _():
                    pltpu.async_copy(
                        tab_hbm.at[pl.ds(stripe * _N_TILES, tail)],
                        acc_sh.at[pl.ds(stripe * _N_TILES, tail)],
                        stsem).wait()
            st.wait()
            plsc.subcore_barrier()

            main = nk - _UNROLL

            @pl.loop(0, main, step=_UNROLL)
            def _(jl):
                for p in range(_UNROLL):
                    j = jl + p
                    pred = True if p >= 2 else (j >= 2)
                    stage(j, p % _RR, p % _IR, pred, True, True)

            for j in range(main, nk):
                stage(j, j % _RR, j % _IR, True, j + 2 < nk, j + 1 < nk)

            wait_scatter((nk - 2) % _RR, (nk - 2) % _IR)
            wait_scatter((nk - 1) % _RR, (nk - 1) % _IR)

            if n_extra:
                @pl.when(sid < n_extra)
                def _():
                    e0 = (nk * _N_TILES + sid) * _EDGE_BLK
                    pltpu.sync_copy(c2_hbm.at[pl.ds(e0, _EDGE_BLK)], ci0)
                    pltpu.sync_copy(u2_hbm.at[pl.ds(e0, _EDGE_BLK)], ui0)
                    pltpu.sync_copy(tab_hbm.at[ci0], rw0)
                    pltpu.sync_copy(rw0, acc_sh.at[ui0], add=True)

            plsc.subcore_barrier()
            pltpu.sync_copy(acc_sh.at[pl.ds(r0, stripe)],
                            out_hbm.at[pl.ds(r0, stripe)])
            if tail:
                @pl.when(sid == _N_TILES - 1)
                def _():
                    pltpu.sync_copy(acc_sh.at[pl.ds(stripe * _N_TILES, tail)],
                                    out_hbm.at[pl.ds(stripe * _N_TILES, tail)])

        @pl.when(cid == 0)
        def _():
            run(hl_hbm, zl_hbm)

        @pl.when(cid == 1)
        def _():
            run(hr_hbm, zr_hbm)

    return agg(hL, hR, c2, u2)


def kernel(x, c_2, u_2, batch, We, be, conv_W1, conv_b1, conv_W2, conv_b2,
           Wr1, br1, Wr2, br2):
    num_graphs = 64
    hL, hR = _encoder(x, We, be)
    for i in range(conv_W1.shape[0]):
        zL, zR = _sc_agg(hL, hR, c_2, u_2)
        hL, hR = _mlp(zL, zR, conv_W1[i], conv_b1[i], conv_W2[i], conv_b2[i])
    return _readout(hL, hR, batch, Wr1, br1, Wr2, br2, num_graphs)
